# Initial kernel scaffold; baseline (speedup 1.0000x reference)
import jax
import jax.numpy as jnp
from jax.experimental import pallas as pl


def kernel(x, learned_table, person_table, num_people):
    raise NotImplementedError("write your pallas kernel here")


# speedup vs baseline: 5.9496x; 5.9496x over previous
import functools

import jax
import jax.numpy as jnp
from jax import lax
from jax.experimental import pallas as pl
from jax.experimental.pallas import tpu as pltpu

B_BLK = 16


def _tc_body(x_ref, le_ref, pe_ref, o_ref):
    le = le_ref[...]
    pe = pe_ref[...]

    def renorm(t):
        ss = jnp.sum(t * t, axis=-1, keepdims=True)
        norm = jnp.sqrt(ss)
        return t * jnp.where(norm > 1.0, 1.0 / (norm + 1e-7), 1.0)

    le = renorm(le)
    pe = renorm(pe)

    row = lax.broadcasted_iota(jnp.int32, (128, 256), 0)
    col = lax.broadcasted_iota(jnp.int32, (128, 256), 1)
    ee = (col == 2 * row).astype(jnp.float32)
    eo = (col == 2 * row + 1).astype(jnp.float32)
    ale = jnp.dot(le, ee, preferred_element_type=jnp.float32)
    ape = jnp.dot(pe, eo, preferred_element_type=jnp.float32)
    add = ale[None, :, None, :] + ape[None, None, :, :]
    o_ref[...] = x_ref[...] + add


def kernel(x, learned_table, person_table, num_people):
    del num_people
    b, s, p, d = x.shape
    pe_rows = lax.slice(person_table, (0, 0), (p, person_table.shape[1]))
    grid = (b // B_BLK,)
    return pl.pallas_call(
        _tc_body,
        grid=grid,
        in_specs=[
            pl.BlockSpec((B_BLK, s, p, d), lambda i: (i, 0, 0, 0)),
            pl.BlockSpec((s, 128), lambda i: (0, 0)),
            pl.BlockSpec((p, 128), lambda i: (0, 0)),
        ],
        out_specs=pl.BlockSpec((B_BLK, s, p, d), lambda i: (i, 0, 0, 0)),
        out_shape=jax.ShapeDtypeStruct(x.shape, x.dtype),
        compiler_params=pltpu.CompilerParams(
            dimension_semantics=("arbitrary",),
        ),
    )(x, learned_table, pe_rows)


# speedup vs baseline: 6.0270x; 1.0130x over previous
import functools

import jax
import jax.numpy as jnp
from jax import lax
from jax.experimental import pallas as pl
from jax.experimental.pallas import tpu as pltpu

B_BLK = 32


def _tc_body(x_ref, le_ref, pe_ref, o_ref):
    le = le_ref[...]
    pe = pe_ref[...]

    def renorm(t):
        ss = jnp.sum(t * t, axis=-1, keepdims=True)
        norm = jnp.sqrt(ss)
        return t * jnp.where(norm > 1.0, 1.0 / (norm + 1e-7), 1.0)

    le = renorm(le)
    pe = renorm(pe)

    row = lax.broadcasted_iota(jnp.int32, (128, 256), 0)
    col = lax.broadcasted_iota(jnp.int32, (128, 256), 1)
    ee = (col == 2 * row).astype(jnp.float32)
    eo = (col == 2 * row + 1).astype(jnp.float32)
    ale = jnp.dot(le, ee, preferred_element_type=jnp.float32)
    ape = jnp.dot(pe, eo, preferred_element_type=jnp.float32)
    add = ale[None, :, None, :] + ape[None, None, :, :]
    o_ref[...] = x_ref[...] + add


def kernel(x, learned_table, person_table, num_people):
    del num_people
    b, s, p, d = x.shape
    pe_rows = lax.slice(person_table, (0, 0), (p, person_table.shape[1]))
    grid = (b // B_BLK,)
    return pl.pallas_call(
        _tc_body,
        grid=grid,
        in_specs=[
            pl.BlockSpec((B_BLK, s, p, d), lambda i: (i, 0, 0, 0)),
            pl.BlockSpec((s, 128), lambda i: (0, 0)),
            pl.BlockSpec((p, 128), lambda i: (0, 0)),
        ],
        out_specs=pl.BlockSpec((B_BLK, s, p, d), lambda i: (i, 0, 0, 0)),
        out_shape=jax.ShapeDtypeStruct(x.shape, x.dtype),
        compiler_params=pltpu.CompilerParams(
            dimension_semantics=("arbitrary",),
        ),
    )(x, learned_table, pe_rows)


# speedup vs baseline: 6.0324x; 1.0009x over previous
import jax
import jax.numpy as jnp
from jax import lax
from jax.experimental import pallas as pl
from jax.experimental.pallas import tpu as pltpu

B_BLK = 32


def _tc_body(x_ref, le_ref, pe_ref, o_ref, a_ref):
    @pl.when(pl.program_id(0) == 0)
    def _build_add():
        le = le_ref[...]
        pe = pe_ref[...]

        def renorm(t):
            ss = jnp.sum(t * t, axis=-1, keepdims=True)
            norm = jnp.sqrt(ss)
            return t * jnp.where(norm > 1.0, 1.0 / (norm + 1e-7), 1.0)

        le = renorm(le)
        pe = renorm(pe)

        row = lax.broadcasted_iota(jnp.int32, (128, 256), 0)
        col = lax.broadcasted_iota(jnp.int32, (128, 256), 1)
        ee = (col == 2 * row).astype(jnp.float32)
        eo = (col == 2 * row + 1).astype(jnp.float32)
        ale = jnp.dot(le, ee, preferred_element_type=jnp.float32)
        ape = jnp.dot(pe, eo, preferred_element_type=jnp.float32)
        a_ref[...] = ale[:, None, :] + ape[None, :, :]

    o_ref[...] = x_ref[...] + a_ref[...][None]


def kernel(x, learned_table, person_table, num_people):
    del num_people
    b, s, p, d = x.shape
    pe_rows = lax.slice(person_table, (0, 0), (p, person_table.shape[1]))
    grid = (b // B_BLK,)
    return pl.pallas_call(
        _tc_body,
        grid=grid,
        in_specs=[
            pl.BlockSpec((B_BLK, s, p, d), lambda i: (i, 0, 0, 0)),
            pl.BlockSpec((s, 128), lambda i: (0, 0)),
            pl.BlockSpec((p, 128), lambda i: (0, 0)),
        ],
        out_specs=pl.BlockSpec((B_BLK, s, p, d), lambda i: (i, 0, 0, 0)),
        out_shape=jax.ShapeDtypeStruct(x.shape, x.dtype),
        scratch_shapes=[pltpu.VMEM((s, p, d), jnp.float32)],
        compiler_params=pltpu.CompilerParams(
            dimension_semantics=("arbitrary",),
        ),
    )(x, learned_table, pe_rows)
